# baseline (speedup 1.0000x reference)
import jax
import jax.numpy as jnp
from jax import lax
from jax.experimental import pallas as pl
from jax.experimental.pallas import tpu as pltpu
from jax.experimental.pallas import tpu_sc as plsc

_B = 16384
_D = 64
_NC = 2
_NS = 16
_NW = _NC * _NS
_BPW = _B // _NW
_CHUNK = 128
_NCHUNK = _BPW // _CHUNK


def _sc_dual_gather(item_table, text_table, idx2d):
    mesh = plsc.VectorSubcoreMesh(core_axis_name="c", subcore_axis_name="s")
    out_t = (
        jax.ShapeDtypeStruct((_B, _D), jnp.float32),
        jax.ShapeDtypeStruct((_B, _D), jnp.float32),
    )

    @pl.kernel(
        out_type=out_t,
        mesh=mesh,
        compiler_params=pltpu.CompilerParams(use_tc_tiling_on_sc=False),
        scratch_types=[
            pltpu.VMEM((_NCHUNK, _CHUNK), jnp.int32),
            pltpu.VMEM((_BPW, _D), jnp.float32),
            pltpu.VMEM((_BPW, _D), jnp.float32),
            pltpu.SemaphoreType.DMA,
        ],
    )
    def k(item_hbm, text_hbm, idx_hbm, oi_hbm, ot_hbm, idx_v, irows, trows, sem):
        wid = lax.axis_index("s") * _NC + lax.axis_index("c")
        base = wid * _BPW
        pltpu.sync_copy(idx_hbm.at[pl.ds(wid * _NCHUNK, _NCHUNK)], idx_v)
        copies = []
        for j in range(_NCHUNK):
            copies.append(pltpu.async_copy(
                item_hbm.at[idx_v.at[j]], irows.at[pl.ds(j * _CHUNK, _CHUNK)], sem))
            copies.append(pltpu.async_copy(
                text_hbm.at[idx_v.at[j]], trows.at[pl.ds(j * _CHUNK, _CHUNK)], sem))
        for c in copies:
            c.wait()
        pltpu.sync_copy(irows, oi_hbm.at[pl.ds(base, _BPW)])
        pltpu.sync_copy(trows, ot_hbm.at[pl.ds(base, _BPW)])

    return k(item_table, text_table, idx2d)


def _mlp_body(item_ref, text_ref, w1_ref, b1_ref, w2_ref, b2_ref, out_ref):
    item = item_ref[...]
    text = text_ref[...]
    w1 = w1_ref[...]
    cdims = (((1,), (1,)), ((), ()))
    h = lax.dot_general(item, w1[:, :_D], cdims,
                        preferred_element_type=jnp.float32)
    h = h + lax.dot_general(text, w1[:, _D:], cdims,
                            preferred_element_type=jnp.float32)
    h = jnp.maximum(h + b1_ref[...], 0.0)
    w2 = w2_ref[...]
    logits = lax.dot_general(h, w2, cdims, preferred_element_type=jnp.float32)
    b2v = b2_ref[...]
    d = (logits[:, 0:1] - logits[:, 1:2]) + (b2v[0, 0] - b2v[0, 1])
    g0 = 1.0 / (1.0 + jnp.exp(-d))
    out_ref[...] = text + g0 * (item - text)


def _tc_gating(item_emb, text_emb, W1, b1, W2, b2, blk=4096):
    grid = (_B // blk,)
    return pl.pallas_call(
        _mlp_body,
        out_shape=jax.ShapeDtypeStruct((_B, _D), jnp.float32),
        grid=grid,
        in_specs=[
            pl.BlockSpec((blk, _D), lambda i: (i, 0)),
            pl.BlockSpec((blk, _D), lambda i: (i, 0)),
            pl.BlockSpec((128, 128), lambda i: (0, 0)),
            pl.BlockSpec((1, 128), lambda i: (0, 0)),
            pl.BlockSpec((2, 128), lambda i: (0, 0)),
            pl.BlockSpec((1, 2), lambda i: (0, 0)),
        ],
        out_specs=pl.BlockSpec((blk, _D), lambda i: (i, 0)),
    )(item_emb, text_emb, W1, b1, W2, b2)


def kernel(item_ids, item_table, text_table, W1, b1, W2, b2):
    idx2d = item_ids.astype(jnp.int32).reshape(_B // _CHUNK, _CHUNK)
    item_emb, text_emb = _sc_dual_gather(item_table, text_table, idx2d)
    return _tc_gating(item_emb, text_emb, W1,
                      b1.reshape(1, 128), W2, b2.reshape(1, 2))


# speedup vs baseline: 1.5825x; 1.5825x over previous
import jax
import jax.numpy as jnp
from jax import lax
from jax.experimental import pallas as pl
from jax.experimental.pallas import tpu as pltpu
from jax.experimental.pallas import tpu_sc as plsc

_B = 16384
_D = 64
_NC = 2
_NS = 16
_NW = _NC * _NS
_BPW = _B // _NW
_CHUNK = 128
_NCHUNK = _BPW // _CHUNK


_PASS = 256


def _sc_dual_gather(item_table, text_table, item_ids):
    mesh = plsc.VectorSubcoreMesh(core_axis_name="c", subcore_axis_name="s")
    out_t = (
        jax.ShapeDtypeStruct((_B, _D), jnp.float32),
        jax.ShapeDtypeStruct((_B, _D), jnp.float32),
    )

    @pl.kernel(
        out_type=out_t,
        mesh=mesh,
        scratch_types=[
            pltpu.VMEM((_BPW,), jnp.int32),
            pltpu.VMEM((_PASS, _D), jnp.float32),
            pltpu.VMEM((_PASS, _D), jnp.float32),
            pltpu.SemaphoreType.DMA,
        ],
    )
    def k(item_hbm, text_hbm, idx_hbm, oi_hbm, ot_hbm, idx_v,
          irows, trows, sem):
        wid = lax.axis_index("s") * _NC + lax.axis_index("c")
        base = wid * _BPW
        pltpu.sync_copy(idx_hbm.at[pl.ds(base, _BPW)], idx_v)
        for p in range(_BPW // _PASS):
            @pl.loop(0, _PASS, step=16)
            def _(j):
                v = idx_v[pl.ds(p * _PASS + j, 16)]
                for q in range(16):
                    r = v[q]
                    pltpu.async_copy(item_hbm.at[pl.ds(r, 1)],
                                     irows.at[pl.ds(j + q, 1)], sem)
                    pltpu.async_copy(text_hbm.at[pl.ds(r, 1)],
                                     trows.at[pl.ds(j + q, 1)], sem)
            pltpu.make_async_copy(item_hbm.at[pl.ds(0, _PASS)], irows, sem).wait()
            pltpu.make_async_copy(text_hbm.at[pl.ds(0, _PASS)], trows, sem).wait()
            pltpu.sync_copy(irows, oi_hbm.at[pl.ds(base + p * _PASS, _PASS)])
            pltpu.sync_copy(trows, ot_hbm.at[pl.ds(base + p * _PASS, _PASS)])

    return k(item_table, text_table, item_ids)


def _mlp_body(item_ref, text_ref, w1_ref, b1_ref, w2_ref, b2_ref, out_ref):
    item = item_ref[...]
    text = text_ref[...]
    w1 = w1_ref[...]
    cdims = (((1,), (1,)), ((), ()))
    h = lax.dot_general(item, w1[:, :_D], cdims,
                        preferred_element_type=jnp.float32)
    h = h + lax.dot_general(text, w1[:, _D:], cdims,
                            preferred_element_type=jnp.float32)
    h = jnp.maximum(h + b1_ref[...], 0.0)
    w2 = w2_ref[...]
    logits = lax.dot_general(h, w2, cdims, preferred_element_type=jnp.float32)
    b2v = b2_ref[...]
    d = (logits[:, 0:1] - logits[:, 1:2]) + (b2v[0, 0] - b2v[0, 1])
    g0 = 1.0 / (1.0 + jnp.exp(-d))
    out_ref[...] = text + g0 * (item - text)


def _tc_gating(item_emb, text_emb, W1, b1, W2, b2, blk=4096):
    grid = (_B // blk,)
    return pl.pallas_call(
        _mlp_body,
        out_shape=jax.ShapeDtypeStruct((_B, _D), jnp.float32),
        grid=grid,
        in_specs=[
            pl.BlockSpec((blk, _D), lambda i: (i, 0)),
            pl.BlockSpec((blk, _D), lambda i: (i, 0)),
            pl.BlockSpec((128, 128), lambda i: (0, 0)),
            pl.BlockSpec((1, 128), lambda i: (0, 0)),
            pl.BlockSpec((2, 128), lambda i: (0, 0)),
            pl.BlockSpec((1, 2), lambda i: (0, 0)),
        ],
        out_specs=pl.BlockSpec((blk, _D), lambda i: (i, 0)),
    )(item_emb, text_emb, W1, b1, W2, b2)


def kernel(item_ids, item_table, text_table, W1, b1, W2, b2):
    item_emb, text_emb = _sc_dual_gather(
        item_table, text_table, item_ids.astype(jnp.int32))
    return _tc_gating(item_emb, text_emb, W1,
                      b1.reshape(1, 128), W2, b2.reshape(1, 2))
